# Initial kernel scaffold; baseline (speedup 1.0000x reference)
import jax
import jax.numpy as jnp
from jax.experimental import pallas as pl


def kernel(ctx_coords, ctx_values, tgt_coords, params):
    raise NotImplementedError("write your pallas kernel here")


# speedup vs baseline: 5.5586x; 5.5586x over previous
import functools

import numpy as np
import jax
import jax.numpy as jnp
from jax.experimental import pallas as pl
from jax.experimental.pallas import tpu as pltpu

_LIN = np.array([
    -14.0, -12.962963104248047, -11.925926208496094, -10.88888931274414,
    -9.851851463317871, -8.814815521240234, -7.777778148651123,
    -6.74074125289917, -5.7037034034729, -4.666666507720947,
    -3.629629611968994, -2.592592477798462, -1.555556058883667,
    -0.5185186266899109, 0.518518328666687, 1.555555820465088,
    2.592592239379883, 3.629629135131836, 4.666666507720947, 5.7037034034729,
    6.740740776062012, 7.777777671813965, 8.814814567565918,
    9.851851463317871, 10.888888359069824, 11.925925254821777,
    12.962963104248047, 14.0,
], dtype=np.float32)

_G = 28
_N = _G * _G
_K = 9
_CHIN = [4, 16, 32, 16]
_COUT = [16, 32, 16, 2]

_GRID = np.stack(np.meshgrid(_LIN, _LIN, indexing='ij'), -1).reshape(_N, 2)

_D2 = ((_GRID[:, None, :] - _GRID[None, :, :]).astype(np.float32) ** 2
       ).sum(-1, dtype=np.float32)
_IDX = np.argsort(_D2, axis=1, kind='stable')[:, :_K]

_OFF_ALL = _IDX - np.arange(_N)[:, None]
_OFFSETS = sorted(set(int(s) for s in np.unique(_OFF_ALL)))
_S = len(_OFFSETS)

_SPAD = 32
_CHCOLS = np.zeros((_N, _SPAD), np.float32)
_REL25 = np.zeros((_SPAD, 2), np.float32)
for _si, _s in enumerate(_OFFSETS):
    _member = (_OFF_ALL == _s).any(axis=1)
    _CHCOLS[:, _si] = _member.astype(np.float32)
    _rows = np.nonzero(_member)[0]
    _rel = (_GRID[_rows + _s] - _GRID[_rows]).astype(np.float64)
    _REL25[_si] = _rel.mean(axis=0).astype(np.float32)


def _expand_mats(c):
    er = np.zeros((c, 16 * c), np.float32)
    et = np.zeros((16, 16 * c), np.float32)
    for cc in range(c):
        er[cc, cc * 16:(cc + 1) * 16] = 1.0
        et[:, cc * 16:(cc + 1) * 16] = np.eye(16, dtype=np.float32)
    return er, et

_EREP = {c: _expand_mats(c)[0] for c in set(_CHIN)}
_ETIL = {c: _expand_mats(c)[1] for c in set(_CHIN)}

_NC_CHUNK = 512
_PAD = 64
_MAXF = 16 * max(_CHIN)
_ROT = np.array([(_PAD * 2 + 784) - (_PAD + s) for s in _OFFSETS], np.int32)


def _body(cx_ref, cy_ref, cv_ref, tgt_ref, tcol_ref, trow_ref, rel_ref,
          ch_ref, sc_ref, offs_ref, e_refs, t_refs, lay_refs, out_ref,
          vpad_ref, part_ref, *, nc, nt):
    tx = tcol_ref[:, 0:1]
    ty = tcol_ref[:, 1:2]
    scale_p = sc_ref[0, 0]
    os_p = sc_ref[0, 1]
    def ctx_body(ci, accs):
        a0, a1 = accs
        sl = pl.ds(ci * _NC_CHUNK, _NC_CHUNK)
        cxc = cx_ref[0, :, sl]
        cyc = cy_ref[0, :, sl]
        cvc = cv_ref[0, :, sl]
        d2 = (tx - cxc) ** 2 + (ty - cyc) ** 2
        kp = os_p * jnp.exp(scale_p * d2)
        a0 = a0 + jnp.sum(kp, axis=1, keepdims=True)
        a1 = a1 + jnp.sum(kp * cvc, axis=1, keepdims=True)
        return (a0, a1)

    acc0, acc1 = jax.lax.fori_loop(
        0, nc // _NC_CHUNK, ctx_body,
        (jnp.zeros((_N, 1), jnp.float32), jnp.zeros((_N, 1), jnp.float32)))
    h0 = acc0
    h1 = acc1 / (acc0 + 1e-8)
    vals = jnp.concatenate([tcol_ref[:, :], h0, h1], axis=1)

    rel25 = rel_ref[:, :]
    chc = ch_ref[:, :]
    vpad_ref[:, :] = jnp.zeros((_N + 2 * _PAD, _MAXF), jnp.float32)
    dot = functools.partial(jnp.dot, preferred_element_type=jnp.float32)
    for li in range(4):
        c, cout = _CHIN[li], _COUT[li]
        f = 16 * c
        w1, b1, w2, b2, w3, b3, lw, lb = lay_refs[li]
        w25 = jnp.maximum(dot(rel25, w1[:, :]) + b1[:, :], 0.0)
        w25 = jnp.maximum(dot(w25, w2[:, :]) + b2[:, :], 0.0)
        w25 = jnp.maximum(dot(w25, w3[:, :]) + b3[:, :], 0.0)
        w25til = dot(w25, t_refs[c][:, :])
        vpad_ref[_PAD:_PAD + _N, 0:f] = dot(vals, e_refs[c][:, :])
        part_ref[:, 0:f] = jnp.zeros((_N, f), jnp.float32)
        vp = vpad_ref[:, 0:f]

        def offs_body(si, carry):
            sel = (jax.lax.broadcasted_iota(jnp.int32, (1, _SPAD), 1)
                   == si).astype(jnp.float32)
            tm = dot(chc * sel, w25til)
            sh = pltpu.roll(vp, offs_ref[si], 0)[0:_N, :]
            part_ref[:, 0:f] = part_ref[:, 0:f] + sh * tm
            return carry

        jax.lax.fori_loop(0, _S, offs_body, jnp.int32(0), unroll=False)
        out = dot(part_ref[:, 0:f] / 9.0, lw[:, :]) + lb[:, :]
        vals = jnp.maximum(out, 0.0) if li < 3 else out

    scale_r = sc_ref[0, 2]
    os_r = sc_ref[0, 3]
    tg = tgt_ref[0]
    d2t = ((tg[:, 0:1] - trow_ref[0:1, :]) ** 2
           + (tg[:, 1:2] - trow_ref[1:2, :]) ** 2)
    kr = os_r * jnp.exp(scale_r * d2t)
    f_sig = vals[:, 1:2]
    sp = jnp.maximum(f_sig, 0.0) + jnp.log1p(jnp.exp(-jnp.abs(f_sig)))
    fcols = jnp.concatenate([vals[:, 0:1], sp], axis=1)
    out_ref[0] = jnp.dot(kr, fcols, preferred_element_type=jnp.float32)


def _kernel_body(*refs, nc, nt):
    (cx, cy, cv, tgt, tcol, trow, rel, ch, sc, offs,
     e4, t4, e16, t16, e32, t32, *rest) = refs
    lay_refs = [tuple(rest[i * 8:(i + 1) * 8]) for i in range(4)]
    out_ref, vpad_ref, part_ref = rest[32:35]
    e_refs = {4: e4, 16: e16, 32: e32}
    t_refs = {4: t4, 16: t16, 32: t32}
    _body(cx, cy, cv, tgt, tcol, trow, rel, ch, sc, offs, e_refs, t_refs,
          lay_refs, out_ref, vpad_ref, part_ref, nc=nc, nt=nt)


def kernel(ctx_coords, ctx_values, tgt_coords, params):
    B, nc, _ = ctx_coords.shape
    nt = tgt_coords.shape[1]

    cx = ctx_coords[..., 0][:, None, :]
    cy = ctx_coords[..., 1][:, None, :]
    cv = ctx_values[..., 0][:, None, :]

    ls_p, os_p = params['ls_psi'], params['os_psi']
    ls_r, os_r = params['ls_rho'], params['os_rho']
    sc = jnp.stack([-0.5 / (ls_p * ls_p), os_p,
                    -0.5 / (ls_r * ls_r), os_r]).reshape(1, 4)

    tcol = jnp.asarray(_GRID)
    trow = jnp.asarray(_GRID.T.copy())
    rel = jnp.asarray(_REL25)
    ch = jnp.asarray(_CHCOLS)

    consts = [jnp.asarray(_EREP[4]), jnp.asarray(_ETIL[4]),
              jnp.asarray(_EREP[16]), jnp.asarray(_ETIL[16]),
              jnp.asarray(_EREP[32]), jnp.asarray(_ETIL[32])]

    lay = []
    for i in range(4):
        lay += [params['w%d_1' % i], params['w%d_1b' % i].reshape(1, -1),
                params['w%d_2' % i], params['w%d_2b' % i].reshape(1, -1),
                params['w%d_3' % i], params['w%d_3b' % i].reshape(1, -1),
                params['lin%d_w' % i], params['lin%d_b' % i].reshape(1, -1)]

    def bspec(shape, batched):
        nd = len(shape)
        if batched:
            blk = (1,) + shape[1:]
            return pl.BlockSpec(blk, lambda b: (b,) + (0,) * (nd - 1))
        return pl.BlockSpec(shape, lambda b: (0,) * nd)

    offs = jnp.asarray(_ROT)

    operands = [cx, cy, cv, tgt_coords, tcol, trow, rel, ch, sc, offs]
    operands += consts + lay
    in_specs = []
    for k, op in enumerate(operands):
        if k == 9:
            in_specs.append(pl.BlockSpec(memory_space=pltpu.SMEM))
        else:
            in_specs.append(bspec(op.shape, batched=(k < 4)))

    musig = pl.pallas_call(
        functools.partial(_kernel_body, nc=nc, nt=nt),
        grid=(B,),
        in_specs=in_specs,
        out_specs=pl.BlockSpec((1, nt, 2), lambda b: (b, 0, 0)),
        out_shape=jax.ShapeDtypeStruct((B, nt, 2), jnp.float32),
        scratch_shapes=[
            pltpu.VMEM((_N + 2 * _PAD, _MAXF), jnp.float32),
            pltpu.VMEM((_N, _MAXF), jnp.float32),
        ],
        compiler_params=pltpu.CompilerParams(
            dimension_semantics=("parallel",)),
    )(*operands)

    mu = musig[..., 0]
    sig = musig[..., 1]
    sigma = sig[:, :, None] * jnp.eye(nt, dtype=jnp.float32)
    return mu, sigma


# speedup vs baseline: 7.2733x; 1.3085x over previous
import functools

import numpy as np
import jax
import jax.numpy as jnp
from jax.experimental import pallas as pl
from jax.experimental.pallas import tpu as pltpu

_LIN = np.array([
    -14.0, -12.962963104248047, -11.925926208496094, -10.88888931274414,
    -9.851851463317871, -8.814815521240234, -7.777778148651123,
    -6.74074125289917, -5.7037034034729, -4.666666507720947,
    -3.629629611968994, -2.592592477798462, -1.555556058883667,
    -0.5185186266899109, 0.518518328666687, 1.555555820465088,
    2.592592239379883, 3.629629135131836, 4.666666507720947, 5.7037034034729,
    6.740740776062012, 7.777777671813965, 8.814814567565918,
    9.851851463317871, 10.888888359069824, 11.925925254821777,
    12.962963104248047, 14.0,
], dtype=np.float32)

_G = 28
_N = _G * _G
_K = 9
_CHIN = [4, 16, 32, 16]
_COUT = [16, 32, 16, 2]

_GRID = np.stack(np.meshgrid(_LIN, _LIN, indexing='ij'), -1).reshape(_N, 2)

_D2 = ((_GRID[:, None, :] - _GRID[None, :, :]).astype(np.float32) ** 2
       ).sum(-1, dtype=np.float32)
_IDX = np.argsort(_D2, axis=1, kind='stable')[:, :_K]

_OFF_ALL = _IDX - np.arange(_N)[:, None]
_OFFSETS = sorted(set(int(s) for s in np.unique(_OFF_ALL)))
_S = len(_OFFSETS)

_SPAD = 32
_CHCOLS = np.zeros((_N, _SPAD), np.float32)
_REL25 = np.zeros((_SPAD, 2), np.float32)
for _si, _s in enumerate(_OFFSETS):
    _member = (_OFF_ALL == _s).any(axis=1)
    _CHCOLS[:, _si] = _member.astype(np.float32)
    _rows = np.nonzero(_member)[0]
    _rel = (_GRID[_rows + _s] - _GRID[_rows]).astype(np.float64)
    _REL25[_si] = _rel.mean(axis=0).astype(np.float32)


def _expand_mats(c):
    er = np.zeros((c, 16 * c), np.float32)
    et = np.zeros((16, 16 * c), np.float32)
    for cc in range(c):
        er[cc, cc * 16:(cc + 1) * 16] = 1.0
        et[:, cc * 16:(cc + 1) * 16] = np.eye(16, dtype=np.float32)
    return er, et

_EREP = {c: _expand_mats(c)[0] for c in set(_CHIN)}
_ETIL = {c: _expand_mats(c)[1] for c in set(_CHIN)}

_NC_CHUNK = 2048
_PAD = 64
_MAXF = 16 * max(_CHIN)
_ROT = np.array([(_PAD * 2 + 784) - (_PAD + s) for s in _OFFSETS], np.int32)


def _body(cx_ref, cy_ref, cv_ref, tgt_ref, tcol_ref, trow_ref, rel_ref,
          ch_ref, sc_ref, offs_ref, e_refs, t_refs, lay_refs, out_ref,
          vpad_ref, part_ref, *, nc, nt):
    tx = tcol_ref[:, 0:1]
    ty = tcol_ref[:, 1:2]
    scale_p = sc_ref[0, 0]
    os_p = sc_ref[0, 1]
    def ctx_body(ci, accs):
        a0, a1 = accs
        sl = pl.ds(ci * _NC_CHUNK, _NC_CHUNK)
        cxc = cx_ref[0, :, sl]
        cyc = cy_ref[0, :, sl]
        cvc = cv_ref[0, :, sl]
        d2 = (tx - cxc) ** 2 + (ty - cyc) ** 2
        kp = os_p * jnp.exp(scale_p * d2)
        a0 = a0 + jnp.sum(kp, axis=1, keepdims=True)
        a1 = a1 + jnp.sum(kp * cvc, axis=1, keepdims=True)
        return (a0, a1)

    acc0, acc1 = jax.lax.fori_loop(
        0, nc // _NC_CHUNK, ctx_body,
        (jnp.zeros((_N, 1), jnp.float32), jnp.zeros((_N, 1), jnp.float32)))
    h0 = acc0
    h1 = acc1 / (acc0 + 1e-8)
    vals = jnp.concatenate([tcol_ref[:, :], h0, h1], axis=1)

    rel25 = rel_ref[:, :]
    chc = ch_ref[:, :]
    vpad_ref[:, :] = jnp.zeros((_N + 2 * _PAD, max(_CHIN)), jnp.float32)
    dot = functools.partial(jnp.dot, preferred_element_type=jnp.float32)
    for li in range(4):
        c, cout = _CHIN[li], _COUT[li]
        f = 16 * c
        w1, b1, w2, b2, w3, b3, lw, lb = lay_refs[li]
        w25 = jnp.maximum(dot(rel25, w1[:, :]) + b1[:, :], 0.0)
        w25 = jnp.maximum(dot(w25, w2[:, :]) + b2[:, :], 0.0)
        w25 = jnp.maximum(dot(w25, w3[:, :]) + b3[:, :], 0.0)
        w25til = dot(w25, t_refs[c][:, :])
        erep = e_refs[c][:, :]
        vpad_ref[_PAD:_PAD + _N, 0:c] = vals
        part_ref[:, 0:f] = jnp.zeros((_N, f), jnp.float32)
        vp = vpad_ref[:, 0:c]

        def offs_body(si, carry):
            sel = (jax.lax.broadcasted_iota(jnp.int32, (1, _SPAD), 1)
                   == si).astype(jnp.float32)
            tm = dot(chc * sel, w25til)
            sh = pltpu.roll(vp, offs_ref[si], 0)[0:_N, :]
            part_ref[:, 0:f] = part_ref[:, 0:f] + dot(sh, erep) * tm
            return carry

        jax.lax.fori_loop(0, _S, offs_body, jnp.int32(0), unroll=False)
        out = dot(part_ref[:, 0:f] / 9.0, lw[:, :]) + lb[:, :]
        vals = jnp.maximum(out, 0.0) if li < 3 else out

    scale_r = sc_ref[0, 2]
    os_r = sc_ref[0, 3]
    tg = tgt_ref[0]
    d2t = ((tg[:, 0:1] - trow_ref[0:1, :]) ** 2
           + (tg[:, 1:2] - trow_ref[1:2, :]) ** 2)
    kr = os_r * jnp.exp(scale_r * d2t)
    f_sig = vals[:, 1:2]
    sp = jnp.maximum(f_sig, 0.0) + jnp.log1p(jnp.exp(-jnp.abs(f_sig)))
    fcols = jnp.concatenate([vals[:, 0:1], sp], axis=1)
    out_ref[0] = jnp.dot(kr, fcols, preferred_element_type=jnp.float32)


def _kernel_body(*refs, nc, nt):
    (cx, cy, cv, tgt, tcol, trow, rel, ch, sc, offs,
     e4, t4, e16, t16, e32, t32, *rest) = refs
    lay_refs = [tuple(rest[i * 8:(i + 1) * 8]) for i in range(4)]
    out_ref, vpad_ref, part_ref = rest[32:35]
    e_refs = {4: e4, 16: e16, 32: e32}
    t_refs = {4: t4, 16: t16, 32: t32}
    _body(cx, cy, cv, tgt, tcol, trow, rel, ch, sc, offs, e_refs, t_refs,
          lay_refs, out_ref, vpad_ref, part_ref, nc=nc, nt=nt)


def kernel(ctx_coords, ctx_values, tgt_coords, params):
    B, nc, _ = ctx_coords.shape
    nt = tgt_coords.shape[1]

    cx = ctx_coords[..., 0][:, None, :]
    cy = ctx_coords[..., 1][:, None, :]
    cv = ctx_values[..., 0][:, None, :]

    ls_p, os_p = params['ls_psi'], params['os_psi']
    ls_r, os_r = params['ls_rho'], params['os_rho']
    sc = jnp.stack([-0.5 / (ls_p * ls_p), os_p,
                    -0.5 / (ls_r * ls_r), os_r]).reshape(1, 4)

    tcol = jnp.asarray(_GRID)
    trow = jnp.asarray(_GRID.T.copy())
    rel = jnp.asarray(_REL25)
    ch = jnp.asarray(_CHCOLS)

    consts = [jnp.asarray(_EREP[4]), jnp.asarray(_ETIL[4]),
              jnp.asarray(_EREP[16]), jnp.asarray(_ETIL[16]),
              jnp.asarray(_EREP[32]), jnp.asarray(_ETIL[32])]

    lay = []
    for i in range(4):
        lay += [params['w%d_1' % i], params['w%d_1b' % i].reshape(1, -1),
                params['w%d_2' % i], params['w%d_2b' % i].reshape(1, -1),
                params['w%d_3' % i], params['w%d_3b' % i].reshape(1, -1),
                params['lin%d_w' % i], params['lin%d_b' % i].reshape(1, -1)]

    def bspec(shape, batched):
        nd = len(shape)
        if batched:
            blk = (1,) + shape[1:]
            return pl.BlockSpec(blk, lambda b: (b,) + (0,) * (nd - 1))
        return pl.BlockSpec(shape, lambda b: (0,) * nd)

    offs = jnp.asarray(_ROT)

    operands = [cx, cy, cv, tgt_coords, tcol, trow, rel, ch, sc, offs]
    operands += consts + lay
    in_specs = []
    for k, op in enumerate(operands):
        if k == 9:
            in_specs.append(pl.BlockSpec(memory_space=pltpu.SMEM))
        else:
            in_specs.append(bspec(op.shape, batched=(k < 4)))

    musig = pl.pallas_call(
        functools.partial(_kernel_body, nc=nc, nt=nt),
        grid=(B,),
        in_specs=in_specs,
        out_specs=pl.BlockSpec((1, nt, 2), lambda b: (b, 0, 0)),
        out_shape=jax.ShapeDtypeStruct((B, nt, 2), jnp.float32),
        scratch_shapes=[
            pltpu.VMEM((_N + 2 * _PAD, max(_CHIN)), jnp.float32),
            pltpu.VMEM((_N, _MAXF), jnp.float32),
        ],
        compiler_params=pltpu.CompilerParams(
            dimension_semantics=("parallel",)),
    )(*operands)

    mu = musig[..., 0]
    sig = musig[..., 1]
    sigma = sig[:, :, None] * jnp.eye(nt, dtype=jnp.float32)
    return mu, sigma


# speedup vs baseline: 8.5199x; 1.1714x over previous
import functools

import numpy as np
import jax
import jax.numpy as jnp
from jax.experimental import pallas as pl
from jax.experimental.pallas import tpu as pltpu

_LIN = np.array([
    -14.0, -12.962963104248047, -11.925926208496094, -10.88888931274414,
    -9.851851463317871, -8.814815521240234, -7.777778148651123,
    -6.74074125289917, -5.7037034034729, -4.666666507720947,
    -3.629629611968994, -2.592592477798462, -1.555556058883667,
    -0.5185186266899109, 0.518518328666687, 1.555555820465088,
    2.592592239379883, 3.629629135131836, 4.666666507720947, 5.7037034034729,
    6.740740776062012, 7.777777671813965, 8.814814567565918,
    9.851851463317871, 10.888888359069824, 11.925925254821777,
    12.962963104248047, 14.0,
], dtype=np.float32)

_G = 28
_N = _G * _G
_K = 9
_CHIN = [4, 16, 32, 16]
_COUT = [16, 32, 16, 2]

_GRID = np.stack(np.meshgrid(_LIN, _LIN, indexing='ij'), -1).reshape(_N, 2)

_D2 = ((_GRID[:, None, :] - _GRID[None, :, :]).astype(np.float32) ** 2
       ).sum(-1, dtype=np.float32)
_IDX = np.argsort(_D2, axis=1, kind='stable')[:, :_K]

_OFF_ALL = _IDX - np.arange(_N)[:, None]
_OFFSETS = sorted(set(int(s) for s in np.unique(_OFF_ALL)))
_S = len(_OFFSETS)

_SPAD = 32
_CHCOLS = np.zeros((_N, _SPAD), np.float32)
_REL25 = np.zeros((_SPAD, 2), np.float32)
for _si, _s in enumerate(_OFFSETS):
    _member = (_OFF_ALL == _s).any(axis=1)
    _CHCOLS[:, _si] = _member.astype(np.float32)
    _rows = np.nonzero(_member)[0]
    _rel = (_GRID[_rows + _s] - _GRID[_rows]).astype(np.float64)
    _REL25[_si] = _rel.mean(axis=0).astype(np.float32)


def _expand_mats(c):
    et = np.zeros((16, 16 * c), np.float32)
    mk = np.zeros((_S * 32, 16 * c), np.float32)
    for cc in range(c):
        et[:, cc * 16:(cc + 1) * 16] = np.eye(16, dtype=np.float32)
        for s in range(_S):
            mk[32 * s + cc, cc * 16:(cc + 1) * 16] = 1.0
    return et, mk

_ETIL = {c: _expand_mats(c)[0] for c in set(_CHIN)}
_MASKC = {c: _expand_mats(c)[1] for c in set(_CHIN)}
_RSEL = np.zeros((_S * 32, _SPAD), np.float32)
for _s2 in range(_S):
    _RSEL[32 * _s2:32 * _s2 + 32, _s2] = 1.0

_NC_CHUNK = 2048
_PAD = 64
_MAXF = 16 * max(_CHIN)
_ROT = np.array([(_PAD * 2 + 784) - (_PAD + s) for s in _OFFSETS], np.int32)


def _body(cx_ref, cy_ref, cv_ref, tgt_ref, tcol_ref, trow_ref, rel_ref,
          ch_ref, sc_ref, offs_ref, rs_ref, m_refs, t_refs, lay_refs, out_ref,
          vpad_ref, part_ref, wl_ref, *, nc, nt):
    tx = tcol_ref[:, 0:1]
    ty = tcol_ref[:, 1:2]
    scale_p = sc_ref[0, 0]
    os_p = sc_ref[0, 1]
    def ctx_body(ci, accs):
        a0, a1 = accs
        sl = pl.ds(ci * _NC_CHUNK, _NC_CHUNK)
        cxc = cx_ref[0, :, sl]
        cyc = cy_ref[0, :, sl]
        cvc = cv_ref[0, :, sl]
        d2 = (tx - cxc) ** 2 + (ty - cyc) ** 2
        kp = os_p * jnp.exp(scale_p * d2)
        a0 = a0 + jnp.sum(kp, axis=1, keepdims=True)
        a1 = a1 + jnp.sum(kp * cvc, axis=1, keepdims=True)
        return (a0, a1)

    acc0, acc1 = jax.lax.fori_loop(
        0, nc // _NC_CHUNK, ctx_body,
        (jnp.zeros((_N, 1), jnp.float32), jnp.zeros((_N, 1), jnp.float32)))
    h0 = acc0
    h1 = acc1 / (acc0 + 1e-8)
    vals = jnp.concatenate([tcol_ref[:, :], h0, h1], axis=1)

    rel25 = rel_ref[:, :]
    chc = ch_ref[:, :]
    vpad_ref[:, :] = jnp.zeros((_N + 2 * _PAD, max(_CHIN)), jnp.float32)
    dot = functools.partial(jnp.dot, preferred_element_type=jnp.float32)
    rsel = rs_ref[:, :]
    for li in range(4):
        c, cout = _CHIN[li], _COUT[li]
        w1, b1, w2, b2, w3, b3, lw, lb = lay_refs[li]
        w25 = jnp.maximum(dot(rel25, w1[:, :]) + b1[:, :], 0.0)
        w25 = jnp.maximum(dot(w25, w2[:, :]) + b2[:, :], 0.0)
        w25 = jnp.maximum(dot(w25, w3[:, :]) + b3[:, :], 0.0)
        wex = dot(rsel, w25)
        wbig = dot(wex, t_refs[c][:, :]) * m_refs[c][:, :]
        wl_ref[:, 0:cout] = dot(wbig, lw[:, :])
        vpad_ref[_PAD:_PAD + _N, 0:c] = vals
        part_ref[:, 0:cout] = jnp.zeros((_N, cout), jnp.float32)
        vp = vpad_ref[:, :]

        def offs_body(si, carry):
            sh = pltpu.roll(vp, offs_ref[si], 0)[0:_N, :]
            l_s = wl_ref[pl.ds(si * 32, 32), 0:cout]
            contrib = dot(sh, l_s)
            selc = (jax.lax.broadcasted_iota(jnp.int32, (_SPAD, 1), 0)
                    == si).astype(jnp.float32)
            chcol = dot(chc, selc)
            part_ref[:, 0:cout] = part_ref[:, 0:cout] + contrib * chcol
            return carry

        jax.lax.fori_loop(0, _S, offs_body, jnp.int32(0), unroll=False)
        out = part_ref[:, 0:cout] / 9.0 + lb[:, :]
        vals = jnp.maximum(out, 0.0) if li < 3 else out

    scale_r = sc_ref[0, 2]
    os_r = sc_ref[0, 3]
    tg = tgt_ref[0]
    d2t = ((tg[:, 0:1] - trow_ref[0:1, :]) ** 2
           + (tg[:, 1:2] - trow_ref[1:2, :]) ** 2)
    kr = os_r * jnp.exp(scale_r * d2t)
    f_sig = vals[:, 1:2]
    sp = jnp.maximum(f_sig, 0.0) + jnp.log1p(jnp.exp(-jnp.abs(f_sig)))
    fcols = jnp.concatenate([vals[:, 0:1], sp], axis=1)
    out_ref[0] = jnp.dot(kr, fcols, preferred_element_type=jnp.float32)


def _kernel_body(*refs, nc, nt):
    (cx, cy, cv, tgt, tcol, trow, rel, ch, sc, offs, rs,
     m4, t4, m16, t16, m32, t32, *rest) = refs
    lay_refs = [tuple(rest[i * 8:(i + 1) * 8]) for i in range(4)]
    out_ref, vpad_ref, part_ref, wl_ref = rest[32:36]
    m_refs = {4: m4, 16: m16, 32: m32}
    t_refs = {4: t4, 16: t16, 32: t32}
    _body(cx, cy, cv, tgt, tcol, trow, rel, ch, sc, offs, rs, m_refs, t_refs,
          lay_refs, out_ref, vpad_ref, part_ref, wl_ref, nc=nc, nt=nt)


def kernel(ctx_coords, ctx_values, tgt_coords, params):
    B, nc, _ = ctx_coords.shape
    nt = tgt_coords.shape[1]

    cx = ctx_coords[..., 0][:, None, :]
    cy = ctx_coords[..., 1][:, None, :]
    cv = ctx_values[..., 0][:, None, :]

    ls_p, os_p = params['ls_psi'], params['os_psi']
    ls_r, os_r = params['ls_rho'], params['os_rho']
    sc = jnp.stack([-0.5 / (ls_p * ls_p), os_p,
                    -0.5 / (ls_r * ls_r), os_r]).reshape(1, 4)

    tcol = jnp.asarray(_GRID)
    trow = jnp.asarray(_GRID.T.copy())
    rel = jnp.asarray(_REL25)
    ch = jnp.asarray(_CHCOLS)

    consts = [jnp.asarray(_RSEL),
              jnp.asarray(_MASKC[4]), jnp.asarray(_ETIL[4]),
              jnp.asarray(_MASKC[16]), jnp.asarray(_ETIL[16]),
              jnp.asarray(_MASKC[32]), jnp.asarray(_ETIL[32])]

    lay = []
    for i in range(4):
        lay += [params['w%d_1' % i], params['w%d_1b' % i].reshape(1, -1),
                params['w%d_2' % i], params['w%d_2b' % i].reshape(1, -1),
                params['w%d_3' % i], params['w%d_3b' % i].reshape(1, -1),
                params['lin%d_w' % i], params['lin%d_b' % i].reshape(1, -1)]

    def bspec(shape, batched):
        nd = len(shape)
        if batched:
            blk = (1,) + shape[1:]
            return pl.BlockSpec(blk, lambda b: (b,) + (0,) * (nd - 1))
        return pl.BlockSpec(shape, lambda b: (0,) * nd)

    offs = jnp.asarray(_ROT)

    operands = [cx, cy, cv, tgt_coords, tcol, trow, rel, ch, sc, offs]
    operands += consts + lay
    in_specs = []
    for k, op in enumerate(operands):
        if k == 9:
            in_specs.append(pl.BlockSpec(memory_space=pltpu.SMEM))
        else:
            in_specs.append(bspec(op.shape, batched=(k < 4)))

    musig = pl.pallas_call(
        functools.partial(_kernel_body, nc=nc, nt=nt),
        grid=(B,),
        in_specs=in_specs,
        out_specs=pl.BlockSpec((1, nt, 2), lambda b: (b, 0, 0)),
        out_shape=jax.ShapeDtypeStruct((B, nt, 2), jnp.float32),
        scratch_shapes=[
            pltpu.VMEM((_N + 2 * _PAD, max(_CHIN)), jnp.float32),
            pltpu.VMEM((_N, max(_COUT)), jnp.float32),
            pltpu.VMEM((_S * 32, max(_COUT)), jnp.float32),
        ],
        compiler_params=pltpu.CompilerParams(
            dimension_semantics=("parallel",)),
    )(*operands)

    mu = musig[..., 0]
    sig = musig[..., 1]
    sigma = sig[:, :, None] * jnp.eye(nt, dtype=jnp.float32)
    return mu, sigma


# speedup vs baseline: 10.0236x; 1.1765x over previous
import functools

import numpy as np
import jax
import jax.numpy as jnp
from jax.experimental import pallas as pl
from jax.experimental.pallas import tpu as pltpu

_LIN = np.array([
    -14.0, -12.962963104248047, -11.925926208496094, -10.88888931274414,
    -9.851851463317871, -8.814815521240234, -7.777778148651123,
    -6.74074125289917, -5.7037034034729, -4.666666507720947,
    -3.629629611968994, -2.592592477798462, -1.555556058883667,
    -0.5185186266899109, 0.518518328666687, 1.555555820465088,
    2.592592239379883, 3.629629135131836, 4.666666507720947, 5.7037034034729,
    6.740740776062012, 7.777777671813965, 8.814814567565918,
    9.851851463317871, 10.888888359069824, 11.925925254821777,
    12.962963104248047, 14.0,
], dtype=np.float32)

_G = 28
_N = _G * _G
_K = 9
_CHIN = [4, 16, 32, 16]
_COUT = [16, 32, 16, 2]

_GRID = np.stack(np.meshgrid(_LIN, _LIN, indexing='ij'), -1).reshape(_N, 2)

_D2 = ((_GRID[:, None, :] - _GRID[None, :, :]).astype(np.float32) ** 2
       ).sum(-1, dtype=np.float32)
_IDX = np.argsort(_D2, axis=1, kind='stable')[:, :_K]

_OFF_ALL = _IDX - np.arange(_N)[:, None]
_OFFSETS = sorted(set(int(s) for s in np.unique(_OFF_ALL)))
_S = len(_OFFSETS)

_SPAD = 32
_CHCOLS = np.zeros((_N, _SPAD), np.float32)
_REL25 = np.zeros((_SPAD, 2), np.float32)
for _si, _s in enumerate(_OFFSETS):
    _member = (_OFF_ALL == _s).any(axis=1)
    _CHCOLS[:, _si] = _member.astype(np.float32)
    _rows = np.nonzero(_member)[0]
    _rel = (_GRID[_rows + _s] - _GRID[_rows]).astype(np.float64)
    _REL25[_si] = _rel.mean(axis=0).astype(np.float32)


def _expand_mats(c):
    et = np.zeros((16, 16 * c), np.float32)
    mk = np.zeros((_S * 32, 16 * c), np.float32)
    for cc in range(c):
        et[:, cc * 16:(cc + 1) * 16] = np.eye(16, dtype=np.float32)
        for s in range(_S):
            mk[32 * s + cc, cc * 16:(cc + 1) * 16] = 1.0
    return et, mk

_ETIL = {c: _expand_mats(c)[0] for c in set(_CHIN)}
_MASKC = {c: _expand_mats(c)[1] for c in set(_CHIN)}
_RSEL = np.zeros((_S * 32, _SPAD), np.float32)
for _s2 in range(_S):
    _RSEL[32 * _s2:32 * _s2 + 32, _s2] = 1.0

_NC_CHUNK = 2048
_PAD = 64
_MAXF = 16 * max(_CHIN)
_PH = np.array([s % 8 for s in _OFFSETS], np.int32)
_B8 = np.array([(s - (s % 8) + _PAD) // 8 for s in _OFFSETS], np.int32)


def _body(cx_ref, cy_ref, cv_ref, tgt_ref, tcol_ref, trow_ref, rel_ref,
          ch_ref, sc_ref, ph_ref, b8_ref, rs_ref, m_refs, t_refs, lay_refs,
          out_ref, vpad_ref, part_ref, wl_ref, *, nc, nt):
    tx = tcol_ref[:, 0:1]
    ty = tcol_ref[:, 1:2]
    scale_p = sc_ref[0, 0]
    os_p = sc_ref[0, 1]
    def ctx_body(ci, accs):
        a0, a1 = accs
        sl = pl.ds(ci * _NC_CHUNK, _NC_CHUNK)
        cxc = cx_ref[0, :, sl]
        cyc = cy_ref[0, :, sl]
        cvc = cv_ref[0, :, sl]
        d2 = (tx - cxc) ** 2 + (ty - cyc) ** 2
        kp = os_p * jnp.exp(scale_p * d2)
        a0 = a0 + jnp.sum(kp, axis=1, keepdims=True)
        a1 = a1 + jnp.sum(kp * cvc, axis=1, keepdims=True)
        return (a0, a1)

    acc0, acc1 = jax.lax.fori_loop(
        0, nc // _NC_CHUNK, ctx_body,
        (jnp.zeros((_N, 1), jnp.float32), jnp.zeros((_N, 1), jnp.float32)))
    h0 = acc0
    h1 = acc1 / (acc0 + 1e-8)
    vals = jnp.concatenate([tcol_ref[:, :], h0, h1], axis=1)

    rel25 = rel_ref[:, :]
    chc = ch_ref[:, :]
    vpad_ref[:, :, :] = jnp.zeros(
        (8, _N + 2 * _PAD, max(_CHIN)), jnp.float32)
    dot = functools.partial(jnp.dot, preferred_element_type=jnp.float32)
    rsel = rs_ref[:, :]
    for li in range(4):
        c, cout = _CHIN[li], _COUT[li]
        w1, b1, w2, b2, w3, b3, lw, lb = lay_refs[li]
        w25 = jnp.maximum(dot(rel25, w1[:, :]) + b1[:, :], 0.0)
        w25 = jnp.maximum(dot(w25, w2[:, :]) + b2[:, :], 0.0)
        w25 = jnp.maximum(dot(w25, w3[:, :]) + b3[:, :], 0.0)
        wex = dot(rsel, w25)
        wbig = dot(wex, t_refs[c][:, :]) * m_refs[c][:, :]
        wl_ref[:, 0:cout] = dot(wbig, lw[:, :])
        for r in range(8):
            vpad_ref[r, _PAD - r:_PAD - r + _N, 0:c] = vals
        part_ref[:, 0:cout] = jnp.zeros((_N, cout), jnp.float32)

        def offs_body(si, carry):
            sh = vpad_ref[ph_ref[si], pl.ds(b8_ref[si] * 8, _N), :]
            l_s = wl_ref[pl.ds(si * 32, 32), 0:cout]
            contrib = dot(sh, l_s)
            selc = (jax.lax.broadcasted_iota(jnp.int32, (_SPAD, 1), 0)
                    == si).astype(jnp.float32)
            chcol = dot(chc, selc)
            part_ref[:, 0:cout] = part_ref[:, 0:cout] + contrib * chcol
            return carry

        jax.lax.fori_loop(0, _S, offs_body, jnp.int32(0), unroll=False)
        out = part_ref[:, 0:cout] / 9.0 + lb[:, :]
        vals = jnp.maximum(out, 0.0) if li < 3 else out

    scale_r = sc_ref[0, 2]
    os_r = sc_ref[0, 3]
    tg = tgt_ref[0]
    d2t = ((tg[:, 0:1] - trow_ref[0:1, :]) ** 2
           + (tg[:, 1:2] - trow_ref[1:2, :]) ** 2)
    kr = os_r * jnp.exp(scale_r * d2t)
    f_sig = vals[:, 1:2]
    sp = jnp.maximum(f_sig, 0.0) + jnp.log1p(jnp.exp(-jnp.abs(f_sig)))
    fcols = jnp.concatenate([vals[:, 0:1], sp], axis=1)
    out_ref[0] = jnp.dot(kr, fcols, preferred_element_type=jnp.float32)


def _kernel_body(*refs, nc, nt):
    (cx, cy, cv, tgt, tcol, trow, rel, ch, sc, ph, b8, rs,
     m4, t4, m16, t16, m32, t32, *rest) = refs
    lay_refs = [tuple(rest[i * 8:(i + 1) * 8]) for i in range(4)]
    out_ref, vpad_ref, part_ref, wl_ref = rest[32:36]
    m_refs = {4: m4, 16: m16, 32: m32}
    t_refs = {4: t4, 16: t16, 32: t32}
    _body(cx, cy, cv, tgt, tcol, trow, rel, ch, sc, ph, b8, rs, m_refs,
          t_refs, lay_refs, out_ref, vpad_ref, part_ref, wl_ref,
          nc=nc, nt=nt)


def kernel(ctx_coords, ctx_values, tgt_coords, params):
    B, nc, _ = ctx_coords.shape
    nt = tgt_coords.shape[1]

    cx = ctx_coords[..., 0][:, None, :]
    cy = ctx_coords[..., 1][:, None, :]
    cv = ctx_values[..., 0][:, None, :]

    ls_p, os_p = params['ls_psi'], params['os_psi']
    ls_r, os_r = params['ls_rho'], params['os_rho']
    sc = jnp.stack([-0.5 / (ls_p * ls_p), os_p,
                    -0.5 / (ls_r * ls_r), os_r]).reshape(1, 4)

    tcol = jnp.asarray(_GRID)
    trow = jnp.asarray(_GRID.T.copy())
    rel = jnp.asarray(_REL25)
    ch = jnp.asarray(_CHCOLS)

    consts = [jnp.asarray(_RSEL),
              jnp.asarray(_MASKC[4]), jnp.asarray(_ETIL[4]),
              jnp.asarray(_MASKC[16]), jnp.asarray(_ETIL[16]),
              jnp.asarray(_MASKC[32]), jnp.asarray(_ETIL[32])]

    lay = []
    for i in range(4):
        lay += [params['w%d_1' % i], params['w%d_1b' % i].reshape(1, -1),
                params['w%d_2' % i], params['w%d_2b' % i].reshape(1, -1),
                params['w%d_3' % i], params['w%d_3b' % i].reshape(1, -1),
                params['lin%d_w' % i], params['lin%d_b' % i].reshape(1, -1)]

    def bspec(shape, batched):
        nd = len(shape)
        if batched:
            blk = (1,) + shape[1:]
            return pl.BlockSpec(blk, lambda b: (b,) + (0,) * (nd - 1))
        return pl.BlockSpec(shape, lambda b: (0,) * nd)

    ph = jnp.asarray(_PH)
    b8 = jnp.asarray(_B8)

    operands = [cx, cy, cv, tgt_coords, tcol, trow, rel, ch, sc, ph, b8]
    operands += consts + lay
    in_specs = []
    for k, op in enumerate(operands):
        if k in (9, 10):
            in_specs.append(pl.BlockSpec(memory_space=pltpu.SMEM))
        else:
            in_specs.append(bspec(op.shape, batched=(k < 4)))

    musig = pl.pallas_call(
        functools.partial(_kernel_body, nc=nc, nt=nt),
        grid=(B,),
        in_specs=in_specs,
        out_specs=pl.BlockSpec((1, nt, 2), lambda b: (b, 0, 0)),
        out_shape=jax.ShapeDtypeStruct((B, nt, 2), jnp.float32),
        scratch_shapes=[
            pltpu.VMEM((8, _N + 2 * _PAD, max(_CHIN)), jnp.float32),
            pltpu.VMEM((_N, max(_COUT)), jnp.float32),
            pltpu.VMEM((_S * 32, max(_COUT)), jnp.float32),
        ],
        compiler_params=pltpu.CompilerParams(
            dimension_semantics=("parallel",)),
    )(*operands)

    mu = musig[..., 0]
    sig = musig[..., 1]
    sigma = sig[:, :, None] * jnp.eye(nt, dtype=jnp.float32)
    return mu, sigma
